# baseline (speedup 1.0000x reference)
import functools

import jax
import jax.numpy as jnp
import numpy as np
from jax import lax
from jax.experimental import pallas as pl
from jax.experimental.pallas import tpu as pltpu
from jax.experimental.pallas import tpu_sc as plsc

_C = 1000
_B = 16384
_STEP_EPOCH = 16
_RSC = 4096
_BTC = _B - _RSC
_NW = 32
_RPW = _RSC // _NW
_L = 16
_TILE = 16
_NCB = _C // _L + 1


def _class_consts():
    ncl = np.array([int(100 * 0.1 ** (i / (_C - 1.0))) for i in range(_C)],
                   dtype=np.float64)
    s = np.log(ncl * (50.0 / ncl.min()))
    s = s * (1.0 / s.min())
    fs = s[::-1]
    m_scale = 0.5 / fs.max()
    a_tab = np.clip(fs, 1.0, 50.0)
    return (s.astype(np.float32)[None, :], np.float32(m_scale),
            a_tab.astype(np.float32))


_S_NP, _M_SCALE, _A_NP = _class_consts()
_K_SCALE = np.float32(np.log(10.0) / (_C - 1.0))
_INV_LOG50 = np.float32(1.0 / np.log(50.0))
_FLOOR_EPS = np.float32(2e-4)
_LOG2E = np.float32(np.log2(np.e))


def _analytic_u(t_f32):
    k = jnp.float32(_C - 1) - t_f32
    v = jnp.float32(100.0) * jnp.exp(-k * _K_SCALE)
    n = jnp.floor(v + _FLOOR_EPS)
    return jnp.log(jnp.float32(5.0) * n) * _INV_LOG50


def _tc_body(ep_ref, t_ref, x_ref, s_ref, o_ref):
    x = x_ref[...]
    t = t_ref[...]
    col = lax.broadcasted_iota(jnp.int32, x.shape, 1)
    oh = col == t
    y = x * s_ref[...]
    yt = jnp.sum(jnp.where(oh, y, 0.0), axis=1, keepdims=True)
    u = _analytic_u(t.astype(jnp.float32))
    a = jnp.clip(u, 1.0, 50.0)
    a = jnp.where(ep_ref[0, 0] < _STEP_EPOCH, jnp.float32(1.0), a)
    a2 = a * _LOG2E
    S = jnp.sum(jnp.exp2(a2 * y), axis=1, keepdims=True)
    ztc = a * (yt - u * _M_SCALE)
    Sc = S - jnp.exp2(a2 * yt) + jnp.exp(ztc)
    nll = jnp.log(Sc) - ztc
    part = jnp.sum(nll, axis=0, keepdims=True) * jnp.float32(1.0 / _B)

    @pl.when(pl.program_id(0) == 0)
    def _init():
        o_ref[...] = jnp.zeros_like(o_ref)

    o_ref[...] += part


def _tc_main(x, t, ep, rows=2048):
    grid = _BTC // rows
    return pl.pallas_call(
        _tc_body,
        grid=(grid,),
        in_specs=[
            pl.BlockSpec(memory_space=pltpu.SMEM),
            pl.BlockSpec((rows, 1), lambda i: (i, 0)),
            pl.BlockSpec((rows, _C), lambda i: (i, 0)),
            pl.BlockSpec((1, _C), lambda i: (0, 0)),
        ],
        out_specs=pl.BlockSpec((1, 1), lambda i: (0, 0)),
        out_shape=jax.ShapeDtypeStruct((1, 1), jnp.float32),
    )(ep, t, x, jnp.asarray(_S_NP))


_RG = 8


def _sc_body(x_hbm, t_hbm, s_hbm, a_hbm, ep_hbm, su_hbm, yt_hbm,
             tv, sv, av, epv, b0, b1, suv, ytv, sem0, sem1, semt):
    info = plsc.get_sparse_core_info()
    wid = lax.axis_index("s") * info.num_cores + lax.axis_index("c")
    row0 = _BTC + wid * _RPW
    pltpu.sync_copy(s_hbm, sv)
    pltpu.sync_copy(ep_hbm, epv)
    pltpu.async_copy(t_hbm.at[pl.ds(row0, _RPW)], tv, semt).wait()
    pltpu.async_copy(a_hbm.at[tv], av, semt).wait()
    ev = epv[...][0]
    lanes = lax.iota(jnp.int32, _L)
    bufs = (b0, b1)
    ntiles = _RPW // _TILE
    cps = [None, None]
    cps[0] = pltpu.async_copy(x_hbm.at[pl.ds(row0, _TILE)], b0, sem0)
    for g in range(ntiles):
        if g + 1 < ntiles:
            nb, ns = bufs[(g + 1) % 2], (sem1 if (g + 1) % 2 else sem0)
            cps[(g + 1) % 2] = pltpu.async_copy(
                x_hbm.at[pl.ds(row0 + (g + 1) * _TILE, _TILE)], nb, ns)
        cps[g % 2].wait()
        buf = bufs[g % 2]
        t16 = tv[pl.ds(g * _TILE, _TILE)]
        a16 = av[pl.ds(g * _TILE, _TILE)]
        for h in range(_TILE // _RG):
            t_s = [t16[h * _RG + j] for j in range(_RG)]
            a_s = [jnp.where(ev < _STEP_EPOCH, jnp.float32(1.0),
                             a16[h * _RG + j]) for j in range(_RG)]

            def cb_body(cb, carry):
                accs, ytas = carry
                off = cb * _L
                sv_c = sv[pl.ds(off, _L)]
                colv = lanes + off
                accs2, ytas2 = [], []
                for j in range(_RG):
                    yv = buf[h * _RG + j, pl.ds(off, _L)] * sv_c
                    e = jnp.exp(yv * a_s[j])
                    accs2.append(accs[j] + e)
                    ytas2.append(jnp.where(colv == t_s[j], yv, ytas[j]))
                return tuple(accs2), tuple(ytas2)

            z16 = tuple(jnp.zeros((_L,), jnp.float32) for _ in range(_RG))
            accs, ytas = lax.fori_loop(0, _NCB - 1, cb_body, (z16, z16))
            off = _C - _L
            sv_c = sv[pl.ds(off, _L)]
            colv = lanes + off
            tmask = lanes >= jnp.int32(_NCB * _L - _C)
            for j in range(_RG):
                r = g * _TILE + h * _RG + j
                yv = buf[h * _RG + j, pl.ds(off, _L)] * sv_c
                e = jnp.where(tmask, jnp.exp(yv * a_s[j]), jnp.float32(0.0))
                suv[r, :] = accs[j] + e
                ytv[r, :] = jnp.where(colv == t_s[j], yv, ytas[j])
    pltpu.sync_copy(suv, su_hbm.at[wid])
    pltpu.sync_copy(ytv, yt_hbm.at[wid])


def _sc_stage(x, t, a_tab, ep_vec):
    mesh = plsc.VectorSubcoreMesh(core_axis_name="c", subcore_axis_name="s")
    f = pl.kernel(
        _sc_body,
        mesh=mesh,
        out_type=[jax.ShapeDtypeStruct((_NW, _RPW, _L), jnp.float32),
                  jax.ShapeDtypeStruct((_NW, _RPW, _L), jnp.float32)],
        scratch_types=[
            pltpu.VMEM((_RPW,), jnp.int32),
            pltpu.VMEM((_C,), jnp.float32),
            pltpu.VMEM((_RPW,), jnp.float32),
            pltpu.VMEM((_L,), jnp.int32),
            pltpu.VMEM((_TILE, _C), jnp.float32),
            pltpu.VMEM((_TILE, _C), jnp.float32),
            pltpu.VMEM((_RPW, _L), jnp.float32),
            pltpu.VMEM((_RPW, _L), jnp.float32),
            pltpu.SemaphoreType.DMA,
            pltpu.SemaphoreType.DMA,
            pltpu.SemaphoreType.DMA,
        ],
    )
    return f(x, t, jnp.asarray(_S_NP[0]), a_tab, ep_vec)


def _epi_body(ep_ref, su_ref, yt_ref, t_ref, main_ref, o_ref):
    t = t_ref[...].astype(jnp.float32)
    u = _analytic_u(t)
    a = jnp.clip(u, 1.0, 50.0)
    a = jnp.where(ep_ref[0, 0] < _STEP_EPOCH, jnp.float32(1.0), a)
    yt = jnp.sum(yt_ref[...], axis=1, keepdims=True)
    S = jnp.sum(su_ref[...], axis=1, keepdims=True)
    ztc = a * (yt - u * _M_SCALE)
    Sc = S - jnp.exp(a * yt) + jnp.exp(ztc)
    nll = jnp.log(Sc) - ztc
    part = jnp.sum(nll, axis=0, keepdims=True) * jnp.float32(1.0 / _B)
    o_ref[...] = main_ref[...] + part


def _epilogue(ep, su, yt, t_sc, main_part):
    return pl.pallas_call(
        _epi_body,
        in_specs=[
            pl.BlockSpec(memory_space=pltpu.SMEM),
            pl.BlockSpec((_RSC, _L), lambda: (0, 0)),
            pl.BlockSpec((_RSC, _L), lambda: (0, 0)),
            pl.BlockSpec((_RSC, 1), lambda: (0, 0)),
            pl.BlockSpec((1, 1), lambda: (0, 0)),
        ],
        out_specs=pl.BlockSpec((1, 1), lambda: (0, 0)),
        out_shape=jax.ShapeDtypeStruct((1, 1), jnp.float32),
    )(ep, su, yt, t_sc, main_part)


@jax.jit
def _kps_loss(x, t, ep, ep_vec):
    su, yt = _sc_stage(x, t, jnp.asarray(_A_NP), ep_vec)
    main = _tc_main(x, t.reshape(_B, 1), ep)
    t_sc = t[_BTC:].reshape(_RSC, 1)
    out = _epilogue(ep, su.reshape(_RSC, _L), yt.reshape(_RSC, _L), t_sc, main)
    return out[0, 0]


def kernel(input, target, epoch):
    t = target.astype(jnp.int32)
    ep = jnp.asarray(epoch, jnp.int32).reshape(1, 1)
    ep_vec = jnp.broadcast_to(jnp.asarray(epoch, jnp.int32), (_L,))
    return _kps_loss(input, t, ep, ep_vec)


# speedup vs baseline: 1.2124x; 1.2124x over previous
import functools

import jax
import jax.numpy as jnp
import numpy as np
from jax.experimental import pallas as pl
from jax.experimental.pallas import tpu as pltpu

_C = 1000
_B = 16384
_STEP_EPOCH = 16


def _class_consts():
    ncl = np.array([int(100 * 0.1 ** (i / (_C - 1.0))) for i in range(_C)],
                   dtype=np.float64)
    s = np.log(ncl * (50.0 / ncl.min()))
    s = s * (1.0 / s.min())
    fs = s[::-1]
    m_scale = 0.5 / fs.max()
    return s.astype(np.float32)[None, :], np.float32(m_scale)


_S_NP, _M_SCALE = _class_consts()
_K_SCALE = np.float32(np.log(10.0) / (_C - 1.0))
_INV_LOG50 = np.float32(1.0 / np.log(50.0))
_FLOOR_EPS = np.float32(2e-4)


def _tc_body(ep_ref, t_ref, x_ref, s_ref, o_ref):
    x = x_ref[...]
    t = t_ref[...]
    col = jax.lax.broadcasted_iota(jnp.int32, x.shape, 1)
    oh = col == t
    y = x * s_ref[...]
    yt = jnp.sum(jnp.where(oh, y, 0.0), axis=1, keepdims=True)
    k = (jnp.int32(_C - 1) - t).astype(jnp.float32)
    v = jnp.float32(100.0) * jnp.exp(-k * _K_SCALE)
    n = jnp.floor(v + _FLOOR_EPS)
    u = jnp.log(jnp.float32(5.0) * n) * _INV_LOG50
    a = jnp.clip(u, 1.0, 50.0)
    a = jnp.where(ep_ref[0, 0] < _STEP_EPOCH, jnp.float32(1.0), a)
    a2 = a * jnp.float32(np.log2(np.e))
    S = jnp.sum(jnp.exp2(a2 * y), axis=1, keepdims=True)
    ztc = a * (yt - u * _M_SCALE)
    Sc = S - jnp.exp2(a2 * yt) + jnp.exp(ztc)
    nll = jnp.log(Sc) - ztc
    part = jnp.sum(nll, axis=0, keepdims=True) * jnp.float32(1.0 / _B)

    @pl.when(pl.program_id(0) == 0)
    def _init():
        o_ref[...] = jnp.zeros_like(o_ref)

    o_ref[...] += part


@functools.partial(jax.jit, static_argnames=("rows",))
def _kps_loss(x, t, ep, rows=1024):
    grid = _B // rows
    out = pl.pallas_call(
        _tc_body,
        grid=(grid,),
        in_specs=[
            pl.BlockSpec(memory_space=pltpu.SMEM),
            pl.BlockSpec((rows, 1), lambda i: (i, 0)),
            pl.BlockSpec((rows, _C), lambda i: (i, 0)),
            pl.BlockSpec((1, _C), lambda i: (0, 0)),
        ],
        out_specs=pl.BlockSpec((1, 1), lambda i: (0, 0)),
        out_shape=jax.ShapeDtypeStruct((1, 1), jnp.float32),
    )(ep, t, x, jnp.asarray(_S_NP))
    return out[0, 0]


def kernel(input, target, epoch):
    t2 = target.astype(jnp.int32).reshape(_B, 1)
    ep = jnp.asarray(epoch, jnp.int32).reshape(1, 1)
    return _kps_loss(input, t2, ep)


# speedup vs baseline: 1.2137x; 1.0010x over previous
import functools

import jax
import jax.numpy as jnp
import numpy as np
from jax.experimental import pallas as pl
from jax.experimental.pallas import tpu as pltpu

_C = 1000
_B = 16384
_STEP_EPOCH = 16


def _class_consts():
    ncl = np.array([int(100 * 0.1 ** (i / (_C - 1.0))) for i in range(_C)],
                   dtype=np.float64)
    s = np.log(ncl * (50.0 / ncl.min()))
    s = s * (1.0 / s.min())
    fs = s[::-1]
    m_scale = 0.5 / fs.max()
    cidx = np.arange(_C, dtype=np.float32)
    return (s.astype(np.float32)[None, :], np.float32(m_scale),
            cidx[None, :])


_S_NP, _M_SCALE, _CIDX_NP = _class_consts()
_K_SCALE = np.float32(np.log(10.0) / (_C - 1.0))
_INV_LOG50 = np.float32(1.0 / np.log(50.0))
_FLOOR_EPS = np.float32(2e-4)
_LOG2E = np.float32(np.log2(np.e))


def _analytic_s(k):
    v = jnp.float32(100.0) * jnp.exp(-k * _K_SCALE)
    n = jnp.floor(v + _FLOOR_EPS)
    return jnp.log(jnp.float32(5.0) * n) * _INV_LOG50


def _tc_body(ep_ref, t_ref, x_ref, s_ref, c_ref, o_ref):
    x = x_ref[...]
    t = t_ref[...]
    oh = c_ref[...] == t
    xt = jnp.sum(jnp.where(oh, x, 0.0), axis=1, keepdims=True)
    st = _analytic_s(t)
    u = _analytic_s(jnp.float32(_C - 1) - t)
    yt = xt * st
    a = jnp.clip(u, 1.0, 50.0)
    a = jnp.where(ep_ref[0, 0] < _STEP_EPOCH, jnp.float32(1.0), a)
    a2 = a * _LOG2E
    S = jnp.sum(jnp.exp2(a2 * (x * s_ref[...])), axis=1, keepdims=True)
    ztc = a * (yt - u * _M_SCALE)
    Sc = S - jnp.exp2(a2 * yt) + jnp.exp(ztc)
    nll = jnp.log(Sc) - ztc
    part = jnp.sum(nll, axis=0, keepdims=True) * jnp.float32(1.0 / _B)

    @pl.when(pl.program_id(0) == 0)
    def _init():
        o_ref[...] = jnp.zeros_like(o_ref)

    o_ref[...] += part


@functools.partial(jax.jit, static_argnames=("rows",))
def _kps_loss(x, t, ep, rows=2048):
    grid = _B // rows
    out = pl.pallas_call(
        _tc_body,
        grid=(grid,),
        in_specs=[
            pl.BlockSpec(memory_space=pltpu.SMEM),
            pl.BlockSpec((rows, 1), lambda i: (i, 0)),
            pl.BlockSpec((rows, _C), lambda i: (i, 0)),
            pl.BlockSpec((1, _C), lambda i: (0, 0)),
            pl.BlockSpec((1, _C), lambda i: (0, 0)),
        ],
        out_specs=pl.BlockSpec((1, 1), lambda i: (0, 0)),
        out_shape=jax.ShapeDtypeStruct((1, 1), jnp.float32),
    )(ep, t, x, jnp.asarray(_S_NP), jnp.asarray(_CIDX_NP))
    return out[0, 0]


def kernel(input, target, epoch):
    t2 = target.astype(jnp.float32).reshape(_B, 1)
    ep = jnp.asarray(epoch, jnp.int32).reshape(1, 1)
    return _kps_loss(input, t2, ep)


# speedup vs baseline: 1.2556x; 1.0345x over previous
import functools

import jax
import jax.numpy as jnp
import numpy as np
from jax.experimental import pallas as pl
from jax.experimental.pallas import tpu as pltpu

_C = 1000
_B = 16384
_STEP_EPOCH = 16


def _class_consts():
    ncl = np.array([int(100 * 0.1 ** (i / (_C - 1.0))) for i in range(_C)],
                   dtype=np.float64)
    s = np.log(ncl * (50.0 / ncl.min()))
    s = s * (1.0 / s.min())
    fs = s[::-1]
    m_scale = 0.5 / fs.max()
    return s.astype(np.float32)[None, :], np.float32(m_scale)


_S_NP, _M_SCALE = _class_consts()
_K_SCALE = np.float32(np.log(10.0) / (_C - 1.0))
_INV_LOG50 = np.float32(1.0 / np.log(50.0))
_FLOOR_EPS = np.float32(2e-4)


def _tc_body(ep_ref, t_ref, x_ref, s_ref, o_ref):
    x = x_ref[...]
    t = t_ref[...]
    col = jax.lax.broadcasted_iota(jnp.int32, x.shape, 1)
    oh = col == t
    y = x * s_ref[...]
    yt = jnp.sum(jnp.where(oh, y, 0.0), axis=1, keepdims=True)
    k = (jnp.int32(_C - 1) - t).astype(jnp.float32)
    v = jnp.float32(100.0) * jnp.exp(-k * _K_SCALE)
    n = jnp.floor(v + _FLOOR_EPS)
    u = jnp.log(jnp.float32(5.0) * n) * _INV_LOG50
    a = jnp.clip(u, 1.0, 50.0)
    a = jnp.where(ep_ref[0, 0] < _STEP_EPOCH, jnp.float32(1.0), a)
    a2 = a * jnp.float32(np.log2(np.e))
    S = jnp.sum(jnp.exp2(a2 * y), axis=1, keepdims=True)
    ztc = a * (yt - u * _M_SCALE)
    Sc = S - jnp.exp2(a2 * yt) + jnp.exp(ztc)
    nll = jnp.log(Sc) - ztc
    part = jnp.sum(nll, axis=0, keepdims=True) * jnp.float32(1.0 / _B)

    @pl.when(pl.program_id(0) == 0)
    def _init():
        o_ref[...] = jnp.zeros_like(o_ref)

    o_ref[...] += part


@functools.partial(jax.jit, static_argnames=("rows",))
def _kps_loss(x, t, ep, rows=2048):
    grid = _B // rows
    out = pl.pallas_call(
        _tc_body,
        grid=(grid,),
        in_specs=[
            pl.BlockSpec(memory_space=pltpu.SMEM),
            pl.BlockSpec((rows, 1), lambda i: (i, 0)),
            pl.BlockSpec((rows, _C), lambda i: (i, 0)),
            pl.BlockSpec((1, _C), lambda i: (0, 0)),
        ],
        out_specs=pl.BlockSpec((1, 1), lambda i: (0, 0)),
        out_shape=jax.ShapeDtypeStruct((1, 1), jnp.float32),
    )(ep, t, x, jnp.asarray(_S_NP))
    return out[0, 0]


def kernel(input, target, epoch):
    t2 = target.astype(jnp.int32).reshape(_B, 1)
    ep = jnp.asarray(epoch, jnp.int32).reshape(1, 1)
    return _kps_loss(input, t2, ep)


# speedup vs baseline: 1.2565x; 1.0008x over previous
import functools

import jax
import jax.numpy as jnp
import numpy as np
from jax.experimental import pallas as pl
from jax.experimental.pallas import tpu as pltpu

_C = 1000
_B = 16384
_STEP_EPOCH = 16


def _class_consts():
    ncl = np.array([int(100 * 0.1 ** (i / (_C - 1.0))) for i in range(_C)],
                   dtype=np.float64)
    s = np.log(ncl * (50.0 / ncl.min()))
    s = s * (1.0 / s.min())
    fs = s[::-1]
    m_scale = 0.5 / fs.max()
    return s.astype(np.float32)[None, :], np.float32(m_scale)


_S_NP, _M_SCALE = _class_consts()
_K_SCALE = np.float32(np.log(10.0) / (_C - 1.0))
_INV_LOG50 = np.float32(1.0 / np.log(50.0))
_FLOOR_EPS = np.float32(2e-4)


def _tc_body(ep_ref, t_ref, x_ref, s_ref, o_ref):
    x = x_ref[...]
    t = t_ref[...][..., None]
    col = jax.lax.broadcasted_iota(jnp.int32, x.shape, 2)
    oh = col == t
    y = x * s_ref[...]
    yt = jnp.sum(jnp.where(oh, y, 0.0), axis=2, keepdims=True)
    k = (jnp.int32(_C - 1) - t).astype(jnp.float32)
    v = jnp.float32(100.0) * jnp.exp(-k * _K_SCALE)
    n = jnp.floor(v + _FLOOR_EPS)
    u = jnp.log(jnp.float32(5.0) * n) * _INV_LOG50
    a = jnp.clip(u, 1.0, 50.0)
    a = jnp.where(ep_ref[0, 0] < _STEP_EPOCH, jnp.float32(1.0), a)
    a2 = a * jnp.float32(np.log2(np.e))
    S = jnp.sum(jnp.exp2(a2 * y), axis=2, keepdims=True)
    ztc = a * (yt - u * _M_SCALE)
    Sc = S - jnp.exp2(a2 * yt) + jnp.exp(ztc)
    nll = jnp.log(Sc[..., 0]) - ztc[..., 0]
    part = jnp.sum(nll, axis=1, keepdims=True)
    part = jnp.sum(part, axis=0, keepdims=True) * jnp.float32(1.0 / _B)

    @pl.when(pl.program_id(0) == 0)
    def _init():
        o_ref[...] = jnp.zeros_like(o_ref)

    o_ref[...] += part


@functools.partial(jax.jit, static_argnames=("rows",))
def _kps_loss(x, t, ep, rows=2048):
    grid = _B // rows
    g = rows // 128
    out = pl.pallas_call(
        _tc_body,
        grid=(grid,),
        in_specs=[
            pl.BlockSpec(memory_space=pltpu.SMEM),
            pl.BlockSpec((g, 128), lambda i: (i, 0)),
            pl.BlockSpec((g, 128, _C), lambda i: (i, 0, 0)),
            pl.BlockSpec((1, _C), lambda i: (0, 0)),
        ],
        out_specs=pl.BlockSpec((1, 1), lambda i: (0, 0)),
        out_shape=jax.ShapeDtypeStruct((1, 1), jnp.float32),
    )(ep, t, x, jnp.asarray(_S_NP))
    return out[0, 0]


def kernel(input, target, epoch):
    t2 = target.astype(jnp.int32).reshape(_B // 128, 128)
    x3 = input.reshape(_B // 128, 128, _C)
    ep = jnp.asarray(epoch, jnp.int32).reshape(1, 1)
    return _kps_loss(x3, t2, ep)


# speedup vs baseline: 4.2498x; 3.3822x over previous
import functools

import jax
import jax.numpy as jnp
import numpy as np
from jax.experimental import pallas as pl
from jax.experimental.pallas import tpu as pltpu

_C = 1000
_B = 16384
_STEP_EPOCH = 16


def _class_consts():
    ncl = np.array([int(100 * 0.1 ** (i / (_C - 1.0))) for i in range(_C)],
                   dtype=np.float64)
    s = np.log(ncl * (50.0 / ncl.min()))
    s = s * (1.0 / s.min())
    fs = s[::-1]
    m_scale = 0.5 / fs.max()
    return s.astype(np.float32)[:, None], np.float32(m_scale)


_S_NP, _M_SCALE = _class_consts()
_K_SCALE = np.float32(np.log(10.0) / (_C - 1.0))
_INV_LOG50 = np.float32(1.0 / np.log(50.0))
_FLOOR_EPS = np.float32(2e-4)
_LOG2E = np.float32(np.log2(np.e))


def _tc_body(ep_ref, t_ref, x_ref, s_ref, o_ref):
    x = x_ref[...]
    t = t_ref[...][0]
    cls = jax.lax.broadcasted_iota(jnp.int32, x.shape, 0)
    oh = cls == t
    y = x * s_ref[...]
    yt = jnp.sum(jnp.where(oh, y, 0.0), axis=0, keepdims=True)
    k = (jnp.int32(_C - 1) - t).astype(jnp.float32)
    v = jnp.float32(100.0) * jnp.exp(-k * _K_SCALE)
    n = jnp.floor(v + _FLOOR_EPS)
    u = jnp.log(jnp.float32(5.0) * n) * _INV_LOG50
    a = jnp.clip(u, 1.0, 50.0)
    a = jnp.where(ep_ref[0, 0] < _STEP_EPOCH, jnp.float32(1.0), a)
    a2 = a * _LOG2E
    S = jnp.sum(jnp.exp2(a2 * y), axis=0, keepdims=True)
    ztc = a * (yt - u * _M_SCALE)
    Sc = S - jnp.exp2(a2 * yt) + jnp.exp(ztc)
    nll = jnp.log(Sc) - ztc
    part = jnp.sum(nll, axis=1, keepdims=True) * jnp.float32(1.0 / _B)

    @pl.when(pl.program_id(0) == 0)
    def _init():
        o_ref[...] = jnp.zeros_like(o_ref)

    o_ref[...] += part


@functools.partial(jax.jit, static_argnames=("cols",))
def _kps_loss(xt, t3, ep, cols=2048):
    grid = _B // cols
    out = pl.pallas_call(
        _tc_body,
        grid=(grid,),
        in_specs=[
            pl.BlockSpec(memory_space=pltpu.SMEM),
            pl.BlockSpec((1, 1, cols), lambda j: (j, 0, 0)),
            pl.BlockSpec((_C, cols), lambda j: (0, j)),
            pl.BlockSpec((_C, 1), lambda j: (0, 0)),
        ],
        out_specs=pl.BlockSpec((1, 1), lambda j: (0, 0)),
        out_shape=jax.ShapeDtypeStruct((1, 1), jnp.float32),
    )(ep, t3, xt, jnp.asarray(_S_NP))
    return out[0, 0]


def kernel(input, target, epoch, cols=2048):
    xt = input.T
    t3 = target.astype(jnp.int32).reshape(_B // cols, 1, cols)
    ep = jnp.asarray(epoch, jnp.int32).reshape(1, 1)
    return _kps_loss(xt, t3, ep, cols=cols)
